# Initial kernel scaffold; baseline (speedup 1.0000x reference)
import jax
import jax.numpy as jnp
from jax.experimental import pallas as pl


def kernel(input, target, impl_l, impl_r, dis_l, dis_r):
    raise NotImplementedError("write your pallas kernel here")


# speedup vs baseline: 2.8192x; 2.8192x over previous
import functools

import jax
import jax.numpy as jnp
from jax import lax
from jax.experimental import pallas as pl
from jax.experimental.pallas import tpu as pltpu
from jax.experimental.pallas import tpu_sc as plsc

B = 256
NL = 1528
NI = 30000
ND = 10000
L = 16
NW = 32
RPW = B // NW


_mesh = plsc.VectorSubcoreMesh(core_axis_name="c", subcore_axis_name="s")


@functools.partial(
    pl.kernel,
    mesh=_mesh,
    compiler_params=pltpu.CompilerParams(needs_layout_passes=False),
    out_type=jax.ShapeDtypeStruct((NW * RPW * 4 * L,), jnp.float32),
    scratch_types=[
        pltpu.VMEM((RPW * NL,), jnp.float32),
        pltpu.VMEM((NI,), jnp.int32),
        pltpu.VMEM((NI,), jnp.int32),
        pltpu.VMEM((ND,), jnp.int32),
        pltpu.VMEM((ND,), jnp.int32),
        pltpu.VMEM((RPW * 4 * L,), jnp.float32),
    ],
)
def _sc_pair_sums(x_hbm, il_hbm, ir_hbm, dl_hbm, dr_hbm, out_hbm,
                  pred_v, il_v, ir_v, dl_v, dr_v, res_v):
    wid = lax.axis_index("s") * 2 + lax.axis_index("c")
    row_words = RPW * NL
    pltpu.sync_copy(x_hbm.at[pl.ds(wid * row_words, row_words)], pred_v)
    pltpu.sync_copy(il_hbm, il_v)
    pltpu.sync_copy(ir_hbm, ir_v)
    pltpu.sync_copy(dl_hbm, dl_v)
    pltpu.sync_copy(dr_hbm, dr_v)

    def sig_body(i, carry):
        xv = pred_v[pl.ds(i * L, L)]
        pred_v[pl.ds(i * L, L)] = 1.0 / (1.0 + jnp.exp(-xv))
        return carry

    lax.fori_loop(0, row_words // L, sig_body, 0)

    def pair_loop(lref, rref, nchunk, disjoint):
        def body(jc, accs):
            ilv = lref[pl.ds(jc * L, L)]
            irv = rref[pl.ds(jc * L, L)]
            out = []
            for b in range(RPW):
                off = b * NL
                lv = plsc.load_gather(pred_v, [ilv + off])
                rv = plsc.load_gather(pred_v, [irv + off])
                if disjoint:
                    c = jnp.maximum(lv + rv - 1.0, 0.0)
                else:
                    c = jnp.maximum(lv - rv, 0.0)
                w = jnp.exp(c)
                out.append(accs[2 * b] + w)
                out.append(accs[2 * b + 1] + w * c)
            return tuple(out)

        zero = jnp.zeros((L,), jnp.float32)
        return lax.fori_loop(0, nchunk, body, tuple(zero for _ in range(2 * RPW)))

    acc_i = pair_loop(il_v, ir_v, NI // L, False)
    acc_d = pair_loop(dl_v, dr_v, ND // L, True)

    for b in range(RPW):
        res_v[pl.ds((b * 4 + 0) * L, L)] = acc_i[2 * b]
        res_v[pl.ds((b * 4 + 1) * L, L)] = acc_i[2 * b + 1]
        res_v[pl.ds((b * 4 + 2) * L, L)] = acc_d[2 * b]
        res_v[pl.ds((b * 4 + 3) * L, L)] = acc_d[2 * b + 1]
    out_words = RPW * 4 * L
    pltpu.sync_copy(res_v, out_hbm.at[pl.ds(wid * out_words, out_words)])


def _tc_combine_body(x_ref, t_ref, s_ref, o_ref):
    x = x_ref[...]
    t = t_ref[...].astype(jnp.float32)
    bce = jnp.sum(
        jnp.maximum(x, 0.0) - x * t + jnp.log1p(jnp.exp(-jnp.abs(x)))
    )
    s = s_ref[...]
    s1i = jnp.sum(s[:, 0 * L:1 * L], axis=1, keepdims=True)
    s2i = jnp.sum(s[:, 1 * L:2 * L], axis=1, keepdims=True)
    s1d = jnp.sum(s[:, 2 * L:3 * L], axis=1, keepdims=True)
    s2d = jnp.sum(s[:, 3 * L:4 * L], axis=1, keepdims=True)
    loss = (bce / (B * NL)
            + jnp.sum(s2i / s1i) / B
            + jnp.sum(s2d / s1d) / B)
    o_ref[...] = jnp.full((1, 1), loss, jnp.float32)


def kernel(input, target, impl_l, impl_r, dis_l, dis_r):
    x = input.astype(jnp.float32)
    sums = _sc_pair_sums(x.reshape(-1), impl_l, impl_r, dis_l, dis_r)
    s = sums.reshape(B, 4 * L)
    out = pl.pallas_call(
        _tc_combine_body,
        out_shape=jax.ShapeDtypeStruct((1, 1), jnp.float32),
    )(x, target, s)
    return out[0, 0]


# speedup vs baseline: 4.3416x; 1.5400x over previous
import functools

import jax
import jax.numpy as jnp
from jax import lax
from jax.experimental import pallas as pl
from jax.experimental.pallas import tpu as pltpu
from jax.experimental.pallas import tpu_sc as plsc

B = 256
NL = 1528
NLP = 1536
NI = 30000
ND = 10000
NIP = 30720
NDP = 10240
L = 16
NW = 32
RPW = B // NW
NP = RPW // 2
FBI = 15
FBD = 8
NSEG = 8

_mesh = plsc.VectorSubcoreMesh(core_axis_name="c", subcore_axis_name="s")


@functools.partial(
    pl.kernel,
    mesh=_mesh,
    compiler_params=pltpu.CompilerParams(needs_layout_passes=False),
    out_type=jax.ShapeDtypeStruct((B, 4 * L), jnp.float32),
    scratch_types=[
        pltpu.VMEM((RPW * NLP,), jnp.float32),
        pltpu.VMEM((NP * NLP,), jnp.int32),
        pltpu.VMEM((NP * NLP,), jnp.int32),
        pltpu.VMEM((NIP,), jnp.int16),
        pltpu.VMEM((NIP,), jnp.int16),
        pltpu.VMEM((NDP,), jnp.int16),
        pltpu.VMEM((NDP,), jnp.int16),
        pltpu.VMEM((RPW, 4 * L), jnp.float32),
        pltpu.SemaphoreType.DMA,
        pltpu.SemaphoreType.DMA,
        pltpu.SemaphoreType.DMA,
    ],
)
def _sc_pair_sums(x_hbm, idx_hbm, out_hbm,
                  xs_v, pp_v, pq_v, il_v, ir_v, dl_v, dr_v, res_v,
                  sem_x, sem_i, sem_d):
    wid = lax.axis_index("s") * 2 + lax.axis_index("c")

    x_cps = [
        pltpu.async_copy(x_hbm.at[pl.ds((wid * RPW + b) * NL, NL)],
                         xs_v.at[pl.ds(b * NLP, NL)], sem_x)
        for b in range(RPW)
    ]
    seg = NIP // NSEG
    seg_cps = []
    for s in range(NSEG):
        lo = s * seg
        seg_cps.append((
            pltpu.async_copy(idx_hbm.at[pl.ds(lo, seg)],
                             il_v.at[pl.ds(lo, seg)], sem_i),
            pltpu.async_copy(idx_hbm.at[pl.ds(NIP + lo, seg)],
                             ir_v.at[pl.ds(lo, seg)], sem_i),
        ))
    d_cps = [
        pltpu.async_copy(idx_hbm.at[pl.ds(2 * NIP, NDP)], dl_v, sem_d),
        pltpu.async_copy(idx_hbm.at[pl.ds(2 * NIP + NDP, NDP)], dr_v, sem_d),
    ]

    nvec = NLP // L

    def sp_block(p, ibase, nun):
        for u in range(nun):
            off = ibase + u * L
            xa = xs_v[pl.ds(2 * p * NLP + off, L)]
            xb = xs_v[pl.ds((2 * p + 1) * NLP + off, L)]
            pa = 1.0 / (1.0 + jnp.exp(-xa))
            pb = 1.0 / (1.0 + jnp.exp(-xb))
            pw = plsc.pack(pa, pb, format=plsc.PackFormat.INTERLEAVED)
            pp_v[pl.ds(p * NLP + off, L)] = plsc.bitcast(pw, jnp.int32)
            qw = plsc.pack(pa - 0.5, pb - 0.5,
                           format=plsc.PackFormat.INTERLEAVED)
            pq_v[pl.ds(p * NLP + off, L)] = plsc.bitcast(qw, jnp.int32)

    for p in range(NP):
        x_cps[2 * p].wait()
        x_cps[2 * p + 1].wait()

        def sp_body(i, carry, _p=p):
            sp_block(_p, i * (L * 4), 4)
            return carry
        lax.fori_loop(0, nvec // 4, sp_body, 0)

    zero32 = jnp.zeros((2 * L,), jnp.bfloat16)

    def pair_block(tab, lref, rref, disjoint, fb):
        def block(bi, mast):
            mast = list(mast)

            def chunk(it, accs):
                accs = list(accs)
                l16 = lref[pl.ds(it * 2 * L, 2 * L)]
                r16 = rref[pl.ds(it * 2 * L, 2 * L)]
                ila, ilb = plsc.unpack(l16, format=plsc.PackFormat.INTERLEAVED)
                ira, irb = plsc.unpack(r16, format=plsc.PackFormat.INTERLEAVED)
                for half, (ilv, irv) in enumerate(((ila, ira), (ilb, irb))):
                    for p in range(NP):
                        tab_p = tab.at[pl.ds(p * NLP, NLP)]
                        wl = plsc.bitcast(
                            plsc.load_gather(tab_p, [ilv]), jnp.bfloat16)
                        wr = plsc.bitcast(
                            plsc.load_gather(tab_p, [irv]), jnp.bfloat16)
                        d = wl + wr if disjoint else wl - wr
                        c = jnp.maximum(d, zero32)
                        w = jnp.exp(c)
                        accs[2 * p] = accs[2 * p] + w
                        accs[2 * p + 1] = accs[2 * p + 1] + w * c
                return tuple(accs)

            accs = lax.fori_loop(bi * fb, (bi + 1) * fb, chunk,
                                 tuple(zero32 for _ in range(2 * NP)))
            for p in range(NP):
                for t in range(2):
                    ev, od = plsc.unpack(accs[2 * p + t],
                                         format=plsc.PackFormat.INTERLEAVED)
                    mast[2 * (2 * p) + t] = mast[2 * (2 * p) + t] + ev
                    mast[2 * (2 * p + 1) + t] = mast[2 * (2 * p + 1) + t] + od
            return tuple(mast)

        return block

    zf = jnp.zeros((L,), jnp.float32)

    blk_i = pair_block(pp_v, il_v, ir_v, False, FBI)
    mi = tuple(zf for _ in range(2 * RPW))
    blocks_per_seg = NIP // (2 * L * FBI) // NSEG
    for s in range(NSEG):
        seg_cps[s][0].wait()
        seg_cps[s][1].wait()
        mi = lax.fori_loop(s * blocks_per_seg, (s + 1) * blocks_per_seg,
                           blk_i, mi)
    mi = list(mi)

    d_cps[0].wait()
    d_cps[1].wait()
    blk_d = pair_block(pq_v, dl_v, dr_v, True, FBD)
    md = list(lax.fori_loop(0, NDP // (2 * L * FBD),
                            blk_d, tuple(zf for _ in range(2 * RPW))))

    for b in range(RPW):
        mi[2 * b] = mi[2 * b] - ((NIP - NI) // L)
    ndf = float((NDP - ND) // L)
    for p in range(NP):
        q0p = plsc.bitcast(
            plsc.load_gather(pq_v, [jnp.full((L,), p * NLP, jnp.int32)]),
            jnp.bfloat16)
        d0p = jnp.maximum(q0p + q0p, zero32)
        w0p = jnp.exp(d0p)
        w0pe, w0po = plsc.unpack(w0p, format=plsc.PackFormat.INTERLEAVED)
        wc0pe, wc0po = plsc.unpack(w0p * d0p,
                                   format=plsc.PackFormat.INTERLEAVED)
        md[2 * (2 * p)] = md[2 * (2 * p)] - ndf * w0pe
        md[2 * (2 * p) + 1] = md[2 * (2 * p) + 1] - ndf * wc0pe
        md[2 * (2 * p + 1)] = md[2 * (2 * p + 1)] - ndf * w0po
        md[2 * (2 * p + 1) + 1] = md[2 * (2 * p + 1) + 1] - ndf * wc0po

    for b in range(RPW):
        res_v[b, pl.ds(0 * L, L)] = mi[2 * b]
        res_v[b, pl.ds(1 * L, L)] = mi[2 * b + 1]
        res_v[b, pl.ds(2 * L, L)] = md[2 * b]
        res_v[b, pl.ds(3 * L, L)] = md[2 * b + 1]
    pltpu.sync_copy(res_v, out_hbm.at[pl.ds(wid * RPW, RPW), :])


def _tc_bce_body(x_ref, t_ref, o_ref):
    x = x_ref[...]
    t = t_ref[...].astype(jnp.float32)
    bce = jnp.sum(
        jnp.maximum(x, 0.0) - x * t + jnp.log1p(jnp.exp(-jnp.abs(x)))
    )
    o_ref[...] = jnp.full((1, 1), bce, jnp.float32)


def _tc_combine_body(b_ref, s_ref, o_ref):
    bce = b_ref[0, 0]
    s = s_ref[...]
    s1i = jnp.sum(s[:, 0 * L:1 * L], axis=1, keepdims=True)
    s2i = jnp.sum(s[:, 1 * L:2 * L], axis=1, keepdims=True)
    s1d = jnp.sum(s[:, 2 * L:3 * L], axis=1, keepdims=True)
    s2d = jnp.sum(s[:, 3 * L:4 * L], axis=1, keepdims=True)
    loss = (bce / (B * NL)
            + jnp.sum(s2i / s1i) / B
            + jnp.sum(s2d / s1d) / B)
    o_ref[...] = jnp.full((1, 1), loss, jnp.float32)


def _pad16(a, n):
    return jnp.concatenate(
        [a, jnp.zeros((n - a.shape[0],), a.dtype)]).astype(jnp.int16)


def kernel(input, target, impl_l, impl_r, dis_l, dis_r):
    x = input.astype(jnp.float32)
    idx = jnp.concatenate([
        _pad16(impl_l, NIP), _pad16(impl_r, NIP),
        _pad16(dis_l, NDP), _pad16(dis_r, NDP),
    ])
    sums = _sc_pair_sums(x.reshape(-1), idx)
    bce = pl.pallas_call(
        _tc_bce_body,
        out_shape=jax.ShapeDtypeStruct((1, 1), jnp.float32),
    )(x, target)
    out = pl.pallas_call(
        _tc_combine_body,
        out_shape=jax.ShapeDtypeStruct((1, 1), jnp.float32),
    )(bce, sums)
    return out[0, 0]
